# baseline (speedup 1.0000x reference)
import functools
import math

import jax
import jax.numpy as jnp
from jax.experimental import pallas as pl
from jax.experimental.pallas import tpu as pltpu

_LOG2E = math.log2(math.e)


def _gat_layer_body(x_ref, adj_ref, w_ref, asbd_ref, adbd_ref, adbd2_ref,
                    o_ref, wh_ref, wht_ref, fs_ref, fd_ref, fdc_ref,
                    r_ref, f_ref, p_ref, whp_ref,
                    *, heads, dh, blk, final):
    i = pl.program_id(0)
    n = wh_ref.shape[0]
    agg = whp_ref.shape[1] // heads

    @pl.when(i == 0)
    def _project():
        xv = x_ref[...]
        wv = w_ref[...]
        wh_ref[...] = jax.lax.dot_general(
            xv, wv, (((1,), (0,)), ((), ())),
            preferred_element_type=jnp.float32)
        wht_ref[...] = jax.lax.dot_general(
            wv, xv, (((0,), (1,)), ((), ())),
            preferred_element_type=jnp.float32)
        fs_ref[...] = jax.lax.dot_general(
            wh_ref[...], asbd_ref[...], (((1,), (0,)), ((), ())),
            preferred_element_type=jnp.float32)
        fd_ref[...] = jax.lax.dot_general(
            adbd_ref[...], wht_ref[...], (((1,), (0,)), ((), ())),
            preferred_element_type=jnp.float32)
        fdc_ref[...] = jax.lax.dot_general(
            wh_ref[...], adbd2_ref[...], (((1,), (0,)), ((), ())),
            preferred_element_type=jnp.float32)
        for h in range(heads):
            sl = slice(h * dh, (h + 1) * dh)
            md = jnp.max(fd_ref[h:h + 1, :], axis=1, keepdims=True)
            r_ref[:, h:h + 1] = jnp.exp2(jnp.clip(
                -0.8 * (fs_ref[:, h:h + 1] + md), -126.0, 126.0))
            f_ref[h:h + 1, :] = jnp.exp2(0.8 * (fd_ref[h:h + 1, :] - md))
            if not final:
                f2c = jnp.exp2(0.2 * (fdc_ref[:, h:h + 1] - md))
                whp_ref[:, h * agg:h * agg + dh] = wh_ref[:, sl] * f2c
                whp_ref[:, h * agg + dh:(h + 1) * agg] = jnp.broadcast_to(
                    f2c, (n, agg - dh))

    adj = adj_ref[...]
    for h in range(heads):
        r_col = r_ref[pl.ds(i * blk, blk), h:h + 1]
        if final:
            fd_row = fd_ref[h:h + 1, :]
            md = jnp.max(fd_row, axis=1, keepdims=True)
            f1 = jnp.exp2(fd_row - md)
            f2 = jnp.exp2(0.2 * (fd_row - md))
            p_ref[:, h * n:(h + 1) * n] = (
                jnp.maximum(f1, r_col * f2) * adj)
        else:
            f_row = f_ref[h:h + 1, :]
            p_ref[:, h * n:(h + 1) * n] = (
                jnp.maximum(f_row, r_col) * adj)
    acc = None
    for h in range(heads):
        p_h = p_ref[:, h * n:(h + 1) * n]
        if final:
            res = jax.lax.dot_general(
                p_h, wh_ref[:, h * dh:(h + 1) * dh],
                (((1,), (0,)), ((), ())),
                preferred_element_type=jnp.float32)
            s = jnp.sum(p_h, axis=1, keepdims=True)
            acc = res / s if acc is None else acc + res / s
        else:
            res = jax.lax.dot_general(
                p_h, whp_ref[:, h * agg:(h + 1) * agg],
                (((1,), (0,)), ((), ())),
                preferred_element_type=jnp.float32)
            out_h = res[:, :dh] / res[:, dh:dh + 1]
            sl = slice(h * dh, (h + 1) * dh)
            elu_neg = jnp.exp(jnp.minimum(out_h, 0.0)) - 1.0
            o_ref[:, sl] = jnp.where(out_h > 0, out_h, elu_neg)
    if final:
        o_ref[...] = acc * jnp.float32(1.0 / heads)


def _gat_layer(x, adj, w_cat, a_s_bd, a_d_bd, a_d_bd2, *, heads, dh, final,
               blk=512):
    n, d_in = x.shape
    hd = heads * dh
    out_dim = dh if final else hd
    agg = 8 if final else dh + 8
    grid = n // blk
    body = functools.partial(_gat_layer_body, heads=heads, dh=dh, blk=blk,
                             final=final)
    return pl.pallas_call(
        body,
        grid=(grid,),
        in_specs=[
            pl.BlockSpec((n, d_in), lambda i: (0, 0)),
            pl.BlockSpec((blk, n), lambda i: (i, 0)),
            pl.BlockSpec((d_in, hd), lambda i: (0, 0)),
            pl.BlockSpec((hd, 8), lambda i: (0, 0)),
            pl.BlockSpec((8, hd), lambda i: (0, 0)),
            pl.BlockSpec((hd, 8), lambda i: (0, 0)),
        ],
        out_specs=pl.BlockSpec((blk, out_dim), lambda i: (i, 0)),
        out_shape=jax.ShapeDtypeStruct((n, out_dim), jnp.float32),
        scratch_shapes=[
            pltpu.VMEM((n, hd), jnp.float32),
            pltpu.VMEM((hd, n), jnp.float32),
            pltpu.VMEM((n, 8), jnp.float32),
            pltpu.VMEM((8, n), jnp.float32),
            pltpu.VMEM((n, 8), jnp.float32),
            pltpu.VMEM((n, 8), jnp.float32),
            pltpu.VMEM((8, n), jnp.float32),
            pltpu.VMEM((blk, heads * n), jnp.float32),
            pltpu.VMEM((n, heads * agg), jnp.float32),
        ],
    )(x, adj, w_cat, a_s_bd, a_d_bd, a_d_bd2)


def _block_diag_attn(a, pad=8):
    heads, dh = a.shape
    eye = jnp.eye(heads, pad, dtype=a.dtype)
    return (_LOG2E * a[:, :, None] * eye[:, None, :]).reshape(heads * dh, pad)


def kernel(x, adj, Ws, a_src, a_dst, Wf, af_src, af_dst):
    h2, dh2 = a_src.shape[1], a_src.shape[2]
    w2 = jnp.transpose(Ws[-1], (1, 0, 2)).reshape(Ws.shape[2], -1)
    ad2 = _block_diag_attn(a_dst[-1])
    y = _gat_layer(x, adj, w2,
                   _block_diag_attn(a_src[-1]),
                   ad2.T, ad2,
                   heads=h2, dh=dh2, final=False, blk=1024)
    hf, dhf = af_src.shape
    wf = jnp.transpose(Wf, (1, 0, 2)).reshape(Wf.shape[1], -1)
    adf = _block_diag_attn(af_dst)
    return _gat_layer(y, adj, wf,
                      _block_diag_attn(af_src),
                      adf.T, adf,
                      heads=hf, dh=dhf, final=True)


# speedup vs baseline: 1.0880x; 1.0880x over previous
import functools
import math

import jax
import jax.numpy as jnp
from jax.experimental import pallas as pl
from jax.experimental.pallas import tpu as pltpu

_LOG2E = math.log2(math.e)


def _gat_layer_body(x_ref, adj_ref, w_ref, asbd_ref, adbd_ref, adbd2_ref,
                    o_ref, wh_ref, wht_ref, fs_ref, fd_ref, fdc_ref,
                    r_ref, f_ref, p_ref, whp_ref,
                    *, heads, dh, blk, final):
    i = pl.program_id(0)
    n = wh_ref.shape[0]
    agg = whp_ref.shape[1] // heads

    @pl.when(i == 0)
    def _project():
        xv = x_ref[...]
        wv = w_ref[...]
        wh_ref[...] = jax.lax.dot_general(
            xv, wv, (((1,), (0,)), ((), ())),
            preferred_element_type=jnp.float32)
        wht_ref[...] = jax.lax.dot_general(
            wv, xv, (((0,), (1,)), ((), ())),
            preferred_element_type=jnp.float32)
        fs_ref[...] = jax.lax.dot_general(
            wh_ref[...], asbd_ref[...], (((1,), (0,)), ((), ())),
            preferred_element_type=jnp.float32)
        fd_ref[...] = jax.lax.dot_general(
            adbd_ref[...], wht_ref[...], (((1,), (0,)), ((), ())),
            preferred_element_type=jnp.float32)
        fdc_ref[...] = jax.lax.dot_general(
            wh_ref[...], adbd2_ref[...], (((1,), (0,)), ((), ())),
            preferred_element_type=jnp.float32)
        for h in range(heads):
            sl = slice(h * dh, (h + 1) * dh)
            md = jnp.max(fd_ref[h:h + 1, :], axis=1, keepdims=True)
            r_ref[:, h:h + 1] = jnp.exp2(jnp.clip(
                -0.8 * (fs_ref[:, h:h + 1] + md), -126.0, 126.0))
            f_ref[h:h + 1, :] = jnp.exp2(0.8 * (fd_ref[h:h + 1, :] - md))
            f2c = jnp.exp2(0.2 * (fdc_ref[:, h:h + 1] - md))
            whp_ref[:, h * agg:h * agg + dh] = (
                wh_ref[:, sl] * f2c).astype(jnp.bfloat16)
            whp_ref[:, h * agg + dh:(h + 1) * agg] = jnp.broadcast_to(
                f2c, (n, agg - dh)).astype(jnp.bfloat16)

    adj = adj_ref[...]
    for h in range(heads):
        r_col = r_ref[pl.ds(i * blk, blk), h:h + 1]
        f_row = f_ref[h:h + 1, :]
        p_ref[:, h * n:(h + 1) * n] = (
            jnp.maximum(f_row, r_col) * adj).astype(jnp.bfloat16)
    acc = None
    for h in range(heads):
        p_h = p_ref[:, h * n:(h + 1) * n]
        res = jax.lax.dot_general(
            p_h, whp_ref[:, h * agg:(h + 1) * agg],
            (((1,), (0,)), ((), ())),
            preferred_element_type=jnp.float32)
        out_h = res[:, :dh] / res[:, dh:dh + 1]
        if final:
            acc = out_h if acc is None else acc + out_h
        else:
            sl = slice(h * dh, (h + 1) * dh)
            elu_neg = jnp.exp(jnp.minimum(out_h, 0.0)) - 1.0
            o_ref[:, sl] = jnp.where(out_h > 0, out_h, elu_neg)
    if final:
        o_ref[...] = acc * jnp.float32(1.0 / heads)


def _gat_layer(x, adj, w_cat, a_s_bd, a_d_bd, a_d_bd2, *, heads, dh, final,
               blk=512):
    n, d_in = x.shape
    hd = heads * dh
    out_dim = dh if final else hd
    agg = dh + 8
    grid = n // blk
    body = functools.partial(_gat_layer_body, heads=heads, dh=dh, blk=blk,
                             final=final)
    return pl.pallas_call(
        body,
        grid=(grid,),
        in_specs=[
            pl.BlockSpec((n, d_in), lambda i: (0, 0)),
            pl.BlockSpec((blk, n), lambda i: (i, 0)),
            pl.BlockSpec((d_in, hd), lambda i: (0, 0)),
            pl.BlockSpec((hd, 8), lambda i: (0, 0)),
            pl.BlockSpec((8, hd), lambda i: (0, 0)),
            pl.BlockSpec((hd, 8), lambda i: (0, 0)),
        ],
        out_specs=pl.BlockSpec((blk, out_dim), lambda i: (i, 0)),
        out_shape=jax.ShapeDtypeStruct((n, out_dim), jnp.float32),
        scratch_shapes=[
            pltpu.VMEM((n, hd), jnp.float32),
            pltpu.VMEM((hd, n), jnp.float32),
            pltpu.VMEM((n, 8), jnp.float32),
            pltpu.VMEM((8, n), jnp.float32),
            pltpu.VMEM((n, 8), jnp.float32),
            pltpu.VMEM((n, 8), jnp.float32),
            pltpu.VMEM((8, n), jnp.float32),
            pltpu.VMEM((blk, heads * n), jnp.bfloat16),
            pltpu.VMEM((n, heads * agg), jnp.bfloat16),
        ],
    )(x, adj, w_cat, a_s_bd, a_d_bd, a_d_bd2)


def _block_diag_attn(a, pad=8):
    heads, dh = a.shape
    eye = jnp.eye(heads, pad, dtype=a.dtype)
    return (_LOG2E * a[:, :, None] * eye[:, None, :]).reshape(heads * dh, pad)


def kernel(x, adj, Ws, a_src, a_dst, Wf, af_src, af_dst):
    h2, dh2 = a_src.shape[1], a_src.shape[2]
    w2 = jnp.transpose(Ws[-1], (1, 0, 2)).reshape(Ws.shape[2], -1)
    ad2 = _block_diag_attn(a_dst[-1])
    y = _gat_layer(x, adj, w2,
                   _block_diag_attn(a_src[-1]),
                   ad2.T, ad2,
                   heads=h2, dh=dh2, final=False)
    hf, dhf = af_src.shape
    wf = jnp.transpose(Wf, (1, 0, 2)).reshape(Wf.shape[1], -1)
    adf = _block_diag_attn(af_dst)
    return _gat_layer(y, adj, wf,
                      _block_diag_attn(af_src),
                      adf.T, adf,
                      heads=hf, dh=dhf, final=True)
